# Initial kernel scaffold; baseline (speedup 1.0000x reference)
import jax
import jax.numpy as jnp
from jax.experimental import pallas as pl


def kernel(raw_x, edge_index, W0, b0, W1, b1, R0, R1):
    raise NotImplementedError("write your pallas kernel here")


# speedup vs baseline: 4.8264x; 4.8264x over previous
import functools

import jax
import jax.numpy as jnp
from jax import lax
from jax.experimental import pallas as pl
from jax.experimental.pallas import tpu as pltpu
from jax.experimental.pallas import tpu_sc as plsc

N = 10000
E = 320000
F = 128
C = 40
NC = 2
NS = 16
B = 80
RB = 1000
D1 = 2 * C

CPT = 640


def _tile_row_base(s):
    return jnp.minimum(s * CPT, N - CPT)

_mesh = plsc.VectorSubcoreMesh(core_axis_name="c", subcore_axis_name="s")
_sc_params = pltpu.CompilerParams(use_tc_tiling_on_sc=False)


def _deg_body(src_hbm, dst_hbm, z16_hbm, out_hbm, acc, idx_v, ones_v):
    c = lax.axis_index("c")
    s = lax.axis_index("s")
    r0 = _tile_row_base(s)

    def fill_ones(i, _):
        ones_v[i, :] = jnp.ones((16,), jnp.float32)
        return 0

    lax.fori_loop(0, B, fill_ones, 0)
    pltpu.sync_copy(z16_hbm.at[pl.ds(r0, CPT)], acc.at[pl.ds(r0, CPT)])
    plsc.subcore_barrier()

    rows_per_tile = (E // B) // NS

    def run(idx_hbm):
        pltpu.sync_copy(idx_hbm.at[s], idx_v)

        def chunk(k, _):
            pltpu.sync_copy(ones_v, acc.at[idx_v.at[k]], add=True)
            return 0

        lax.fori_loop(0, rows_per_tile, chunk, 0)

    @pl.when(c == 0)
    def _():
        run(src_hbm)

    @pl.when(c == 1)
    def _():
        run(dst_hbm)

    plsc.subcore_barrier()
    pltpu.sync_copy(acc.at[pl.ds(r0, CPT)], out_hbm.at[c, pl.ds(r0, CPT)])


_deg_call = pl.kernel(
    _deg_body,
    out_type=jax.ShapeDtypeStruct((NC, N, 16), jnp.float32),
    mesh=_mesh,
    compiler_params=_sc_params,
    scratch_types=[
        pltpu.VMEM_SHARED((N, 16), jnp.float32),
        pltpu.VMEM(((E // B) // NS, B), jnp.int32),
        pltpu.VMEM((B, 16), jnp.float32),
    ],
)


FH = F // 2


def _spmm0_body(f0_hbm, f1_hbm, f2_hbm, f3_hbm, src_hbm, dst_hbm, zf_hbm,
                o0_hbm, o1_hbm, o2_hbm, o3_hbm, acc, sidx, didx, rows, sem):
    c = lax.axis_index("c")
    s = lax.axis_index("s")
    r0 = _tile_row_base(s)
    rows_per_tile = (E // B) // NS
    pltpu.sync_copy(src_hbm.at[s], sidx)
    pltpu.sync_copy(dst_hbm.at[s], didx)

    def one_pass(feat_hbm, out_hbm):
        pltpu.sync_copy(zf_hbm.at[pl.ds(r0, CPT)], acc.at[pl.ds(r0, CPT)])
        plsc.subcore_barrier()

        def chunk(k, _):
            pltpu.async_copy(feat_hbm.at[sidx.at[k]], rows, sem).wait()
            pltpu.sync_copy(rows, acc.at[didx.at[k]], add=True)
            return 0

        lax.fori_loop(0, rows_per_tile, chunk, 0)
        plsc.subcore_barrier()
        pltpu.sync_copy(acc.at[pl.ds(r0, CPT)], out_hbm.at[pl.ds(r0, CPT)])
        plsc.subcore_barrier()

    @pl.when(c == 0)
    def _():
        one_pass(f0_hbm, o0_hbm)
        one_pass(f1_hbm, o1_hbm)

    @pl.when(c == 1)
    def _():
        one_pass(f2_hbm, o2_hbm)
        one_pass(f3_hbm, o3_hbm)


_spmm0_call = pl.kernel(
    _spmm0_body,
    out_type=tuple(jax.ShapeDtypeStruct((N, FH), jnp.float32) for _ in range(4)),
    mesh=_mesh,
    compiler_params=_sc_params,
    scratch_types=[
        pltpu.VMEM_SHARED((N, FH), jnp.float32),
        pltpu.VMEM(((E // B) // NS, B), jnp.int32),
        pltpu.VMEM(((E // B) // NS, B), jnp.int32),
        pltpu.VMEM((B, FH), jnp.float32),
        pltpu.SemaphoreType.DMA,
    ],
)


def _spmm1_body(cat_hbm, src_hbm, dst_hbm, zd_hbm, out_hbm,
                acc, sidx, didx, rows, sem):
    c = lax.axis_index("c")
    s = lax.axis_index("s")
    r0 = _tile_row_base(s)
    pltpu.sync_copy(zd_hbm.at[pl.ds(r0, CPT)], acc.at[pl.ds(r0, CPT)])
    plsc.subcore_barrier()

    rows_per_worker = (E // B) // (NC * NS)
    wid = c * NS + s
    pltpu.sync_copy(src_hbm.at[wid], sidx)
    pltpu.sync_copy(dst_hbm.at[wid], didx)

    def chunk(k, _):
        pltpu.async_copy(cat_hbm.at[sidx.at[k]], rows, sem).wait()
        pltpu.sync_copy(rows, acc.at[didx.at[k]], add=True)
        return 0

    lax.fori_loop(0, rows_per_worker, chunk, 0)
    plsc.subcore_barrier()
    pltpu.sync_copy(acc.at[pl.ds(r0, CPT)], out_hbm.at[c, pl.ds(r0, CPT)])


_spmm1_call = pl.kernel(
    _spmm1_body,
    out_type=jax.ShapeDtypeStruct((NC, N, D1), jnp.float32),
    mesh=_mesh,
    compiler_params=_sc_params,
    scratch_types=[
        pltpu.VMEM_SHARED((N, D1), jnp.float32),
        pltpu.VMEM(((E // B) // (NC * NS), B), jnp.int32),
        pltpu.VMEM(((E // B) // (NC * NS), B), jnp.int32),
        pltpu.VMEM((B, D1), jnp.float32),
        pltpu.SemaphoreType.DMA,
    ],
)


def _pre_body(cnt_ref, x_ref, w0_ref, r0_ref, r1_ref,
              f0_ref, f1_ref, f2_ref, f3_ref, z1_ref, dio_ref, dii_ref):
    cnt = cnt_ref[...]
    dout = lax.rsqrt(jnp.maximum(jnp.max(cnt[0], axis=1, keepdims=True), 1.0))
    din = lax.rsqrt(jnp.maximum(jnp.max(cnt[1], axis=1, keepdims=True), 1.0))
    x = x_ref[...]
    z0 = jnp.dot(x, r0_ref[...], preferred_element_type=jnp.float32)
    fg = jnp.dot(x, w0_ref[...], preferred_element_type=jnp.float32) * dout
    f0_ref[...] = fg[:, :FH]
    f1_ref[...] = fg[:, FH:]
    f2_ref[...] = z0[:, :FH]
    f3_ref[...] = z0[:, FH:]
    z1_ref[...] = jnp.dot(z0, r1_ref[...], preferred_element_type=jnp.float32)
    dio_ref[...] = dout
    dii_ref[...] = din


def _pre_call(counts, x, W0, R0, R1):
    return pl.pallas_call(
        _pre_body,
        grid=(N // RB,),
        in_specs=[
            pl.BlockSpec((NC, RB, 16), lambda i: (0, i, 0)),
            pl.BlockSpec((RB, F), lambda i: (i, 0)),
            pl.BlockSpec((F, F), lambda i: (0, 0)),
            pl.BlockSpec((F, F), lambda i: (0, 0)),
            pl.BlockSpec((F, C), lambda i: (0, 0)),
        ],
        out_specs=[
            pl.BlockSpec((RB, FH), lambda i: (i, 0)),
            pl.BlockSpec((RB, FH), lambda i: (i, 0)),
            pl.BlockSpec((RB, FH), lambda i: (i, 0)),
            pl.BlockSpec((RB, FH), lambda i: (i, 0)),
            pl.BlockSpec((RB, C), lambda i: (i, 0)),
            pl.BlockSpec((RB, 1), lambda i: (i, 0)),
            pl.BlockSpec((RB, 1), lambda i: (i, 0)),
        ],
        out_shape=[
            jax.ShapeDtypeStruct((N, FH), jnp.float32),
            jax.ShapeDtypeStruct((N, FH), jnp.float32),
            jax.ShapeDtypeStruct((N, FH), jnp.float32),
            jax.ShapeDtypeStruct((N, FH), jnp.float32),
            jax.ShapeDtypeStruct((N, C), jnp.float32),
            jax.ShapeDtypeStruct((N, 1), jnp.float32),
            jax.ShapeDtypeStruct((N, 1), jnp.float32),
        ],
    )(counts, x, W0, R0, R1)


def _mid_body(s0_ref, s1_ref, s2_ref, s3_ref, dii_ref, dio_ref, b0_ref,
              w1_ref, z1_ref, cat_ref):
    sg = jnp.concatenate([s0_ref[...], s1_ref[...]], axis=1)
    sz = jnp.concatenate([s2_ref[...], s3_ref[...]], axis=1)
    x1 = jnp.maximum(sg * dii_ref[...] + b0_ref[...] + sz, 0.0)
    h1 = jnp.dot(x1, w1_ref[...], preferred_element_type=jnp.float32) * dio_ref[...]
    cat_ref[...] = jnp.concatenate([h1, z1_ref[...]], axis=1)


def _mid_call(s0, s1, s2, s3, dii, dio, b0, W1, z1):
    return pl.pallas_call(
        _mid_body,
        grid=(N // RB,),
        in_specs=[
            pl.BlockSpec((RB, FH), lambda i: (i, 0)),
            pl.BlockSpec((RB, FH), lambda i: (i, 0)),
            pl.BlockSpec((RB, FH), lambda i: (i, 0)),
            pl.BlockSpec((RB, FH), lambda i: (i, 0)),
            pl.BlockSpec((RB, 1), lambda i: (i, 0)),
            pl.BlockSpec((RB, 1), lambda i: (i, 0)),
            pl.BlockSpec((1, F), lambda i: (0, 0)),
            pl.BlockSpec((F, C), lambda i: (0, 0)),
            pl.BlockSpec((RB, C), lambda i: (i, 0)),
        ],
        out_specs=pl.BlockSpec((RB, D1), lambda i: (i, 0)),
        out_shape=jax.ShapeDtypeStruct((N, D1), jnp.float32),
    )(s0, s1, s2, s3, dii, dio, b0, W1, z1)


def _final_body(p_ref, dii_ref, b1_ref, y_ref):
    p = p_ref[...]
    s_h = p[0, :, 0, :] + p[1, :, 0, :]
    s_z = p[0, :, 1, :] + p[1, :, 1, :]
    y_ref[...] = s_h * dii_ref[...] + b1_ref[...] + s_z


def _final_call(parts4, dii, b1):
    return pl.pallas_call(
        _final_body,
        grid=(N // RB,),
        in_specs=[
            pl.BlockSpec((NC, RB, 2, C), lambda i: (0, i, 0, 0)),
            pl.BlockSpec((RB, 1), lambda i: (i, 0)),
            pl.BlockSpec((1, C), lambda i: (0, 0)),
        ],
        out_specs=pl.BlockSpec((RB, C), lambda i: (i, 0)),
        out_shape=jax.ShapeDtypeStruct((N, C), jnp.float32),
    )(parts4, dii, b1)


def kernel(raw_x, edge_index, W0, b0, W1, b1, R0, R1):
    src = edge_index[0].astype(jnp.int32)
    dst = edge_index[1].astype(jnp.int32)
    src16 = src.reshape(NS, (E // B) // NS, B)
    dst16 = dst.reshape(NS, (E // B) // NS, B)
    src32 = src.reshape(NC * NS, (E // B) // (NC * NS), B)
    dst32 = dst.reshape(NC * NS, (E // B) // (NC * NS), B)

    counts = _deg_call(src16, dst16, jnp.zeros((N, 16), jnp.float32))
    f0, f1, f2, f3, z1, dio, dii = _pre_call(counts, raw_x, W0, R0, R1)
    s0, s1, s2, s3 = _spmm0_call(f0, f1, f2, f3, src16, dst16,
                                 jnp.zeros((N, FH), jnp.float32))
    cat1 = _mid_call(s0, s1, s2, s3, dii, dio, b0.reshape(1, F), W1, z1)
    parts = _spmm1_call(cat1, src32, dst32, jnp.zeros((N, D1), jnp.float32))
    y = _final_call(parts.reshape(NC, N, 2, C), dii, b1.reshape(1, C))
    return y


# speedup vs baseline: 7.4296x; 1.5394x over previous
import functools

import jax
import jax.numpy as jnp
from jax import lax
from jax.experimental import pallas as pl
from jax.experimental.pallas import tpu as pltpu
from jax.experimental.pallas import tpu_sc as plsc

N = 10000
E = 320000
F = 128
C = 40
NC = 2
NS = 16
B = 80
RB = 1000
D1 = 2 * C

CPT = 640


def _tile_row_base(s):
    return jnp.minimum(s * CPT, N - CPT)

_mesh = plsc.VectorSubcoreMesh(core_axis_name="c", subcore_axis_name="s")
_sc_params = pltpu.CompilerParams(use_tc_tiling_on_sc=False)


def _deg_body(src_hbm, dst_hbm, z16_hbm, out_hbm, acc, idx_v, ones_v):
    c = lax.axis_index("c")
    s = lax.axis_index("s")
    r0 = _tile_row_base(s)

    def fill_ones(i, _):
        ones_v[i, :] = jnp.ones((16,), jnp.float32)
        return 0

    lax.fori_loop(0, B, fill_ones, 0)
    pltpu.sync_copy(z16_hbm.at[pl.ds(r0, CPT)], acc.at[pl.ds(r0, CPT)])
    plsc.subcore_barrier()

    rows_per_tile = (E // B) // NS

    def run(idx_hbm):
        pltpu.sync_copy(idx_hbm.at[s], idx_v)

        def chunk(k, _):
            pltpu.sync_copy(ones_v, acc.at[idx_v.at[k]], add=True)
            return 0

        lax.fori_loop(0, rows_per_tile, chunk, 0)

    @pl.when(c == 0)
    def _():
        run(src_hbm)

    @pl.when(c == 1)
    def _():
        run(dst_hbm)

    plsc.subcore_barrier()
    pltpu.sync_copy(acc.at[pl.ds(r0, CPT)], out_hbm.at[c, pl.ds(r0, CPT)])


_deg_call = pl.kernel(
    _deg_body,
    out_type=jax.ShapeDtypeStruct((NC, N, 16), jnp.float32),
    mesh=_mesh,
    compiler_params=_sc_params,
    scratch_types=[
        pltpu.VMEM_SHARED((N, 16), jnp.float32),
        pltpu.VMEM(((E // B) // NS, B), jnp.int32),
        pltpu.VMEM((B, 16), jnp.float32),
    ],
)


FH = F // 2


def _spmm_pipeline(feat_hbm, acc, sidx, didx, rows0, rows1, g0, g1, nchunks):
    pltpu.async_copy(feat_hbm.at[sidx.at[0]], rows0, g0)
    pltpu.async_copy(feat_hbm.at[sidx.at[1]], rows1, g1)

    def body(p, _):
        k0 = 2 * p
        pltpu.make_async_copy(feat_hbm.at[sidx.at[k0]], rows0, g0).wait()
        pltpu.sync_copy(rows0, acc.at[didx.at[k0]], add=True)

        @pl.when(k0 + 2 < nchunks)
        def _():
            pltpu.async_copy(feat_hbm.at[sidx.at[k0 + 2]], rows0, g0)

        k1 = k0 + 1
        pltpu.make_async_copy(feat_hbm.at[sidx.at[k1]], rows1, g1).wait()
        pltpu.sync_copy(rows1, acc.at[didx.at[k1]], add=True)

        @pl.when(k1 + 2 < nchunks)
        def _():
            pltpu.async_copy(feat_hbm.at[sidx.at[k1 + 2]], rows1, g1)

        return 0

    lax.fori_loop(0, nchunks // 2, body, 0)
    if nchunks % 2:
        k = nchunks - 1
        pltpu.make_async_copy(feat_hbm.at[sidx.at[k]], rows0, g0).wait()
        pltpu.sync_copy(rows0, acc.at[didx.at[k]], add=True)


def _spmm0_body(f0_hbm, f1_hbm, f2_hbm, f3_hbm, src_hbm, dst_hbm, zf_hbm,
                o0_hbm, o1_hbm, o2_hbm, o3_hbm,
                acc, sidx, didx, rows0, rows1, g0, g1):
    c = lax.axis_index("c")
    s = lax.axis_index("s")
    r0 = _tile_row_base(s)
    rows_per_tile = (E // B) // NS
    pltpu.sync_copy(src_hbm.at[s], sidx)
    pltpu.sync_copy(dst_hbm.at[s], didx)

    def one_pass(feat_hbm, out_hbm):
        pltpu.sync_copy(zf_hbm.at[pl.ds(r0, CPT)], acc.at[pl.ds(r0, CPT)])
        plsc.subcore_barrier()
        _spmm_pipeline(feat_hbm, acc, sidx, didx, rows0, rows1, g0, g1,
                       rows_per_tile)
        plsc.subcore_barrier()
        pltpu.sync_copy(acc.at[pl.ds(r0, CPT)], out_hbm.at[pl.ds(r0, CPT)])
        plsc.subcore_barrier()

    @pl.when(c == 0)
    def _():
        one_pass(f0_hbm, o0_hbm)
        one_pass(f1_hbm, o1_hbm)

    @pl.when(c == 1)
    def _():
        one_pass(f2_hbm, o2_hbm)
        one_pass(f3_hbm, o3_hbm)


_spmm0_call = pl.kernel(
    _spmm0_body,
    out_type=tuple(jax.ShapeDtypeStruct((N, FH), jnp.float32) for _ in range(4)),
    mesh=_mesh,
    compiler_params=_sc_params,
    scratch_types=[
        pltpu.VMEM_SHARED((N, FH), jnp.float32),
        pltpu.VMEM(((E // B) // NS, B), jnp.int32),
        pltpu.VMEM(((E // B) // NS, B), jnp.int32),
        pltpu.VMEM((B, FH), jnp.float32),
        pltpu.VMEM((B, FH), jnp.float32),
        pltpu.SemaphoreType.DMA,
        pltpu.SemaphoreType.DMA,
    ],
)


def _spmm1_body(cat_hbm, src_hbm, dst_hbm, zd_hbm, out_hbm,
                acc, sidx, didx, rows0, rows1, g0, g1):
    c = lax.axis_index("c")
    s = lax.axis_index("s")
    r0 = _tile_row_base(s)
    pltpu.sync_copy(zd_hbm.at[pl.ds(r0, CPT)], acc.at[pl.ds(r0, CPT)])
    plsc.subcore_barrier()

    rows_per_worker = (E // B) // (NC * NS)
    wid = c * NS + s
    pltpu.sync_copy(src_hbm.at[wid], sidx)
    pltpu.sync_copy(dst_hbm.at[wid], didx)

    _spmm_pipeline(cat_hbm, acc, sidx, didx, rows0, rows1, g0, g1,
                   rows_per_worker)
    plsc.subcore_barrier()
    pltpu.sync_copy(acc.at[pl.ds(r0, CPT)], out_hbm.at[c, pl.ds(r0, CPT)])


_spmm1_call = pl.kernel(
    _spmm1_body,
    out_type=jax.ShapeDtypeStruct((NC, N, D1), jnp.float32),
    mesh=_mesh,
    compiler_params=_sc_params,
    scratch_types=[
        pltpu.VMEM_SHARED((N, D1), jnp.float32),
        pltpu.VMEM(((E // B) // (NC * NS), B), jnp.int32),
        pltpu.VMEM(((E // B) // (NC * NS), B), jnp.int32),
        pltpu.VMEM((B, D1), jnp.float32),
        pltpu.VMEM((B, D1), jnp.float32),
        pltpu.SemaphoreType.DMA,
        pltpu.SemaphoreType.DMA,
    ],
)


def _pre_body(cnt_ref, x_ref, w0_ref, r0_ref, r1_ref,
              f0_ref, f1_ref, f2_ref, f3_ref, z1_ref, dio_ref, dii_ref):
    cnt = cnt_ref[...]
    dout = lax.rsqrt(jnp.maximum(jnp.max(cnt[0], axis=1, keepdims=True), 1.0))
    din = lax.rsqrt(jnp.maximum(jnp.max(cnt[1], axis=1, keepdims=True), 1.0))
    x = x_ref[...]
    z0 = jnp.dot(x, r0_ref[...], preferred_element_type=jnp.float32)
    fg = jnp.dot(x, w0_ref[...], preferred_element_type=jnp.float32) * dout
    f0_ref[...] = fg[:, :FH]
    f1_ref[...] = fg[:, FH:]
    f2_ref[...] = z0[:, :FH]
    f3_ref[...] = z0[:, FH:]
    z1_ref[...] = jnp.dot(z0, r1_ref[...], preferred_element_type=jnp.float32)
    dio_ref[...] = dout
    dii_ref[...] = din


def _pre_call(counts, x, W0, R0, R1):
    return pl.pallas_call(
        _pre_body,
        grid=(N // RB,),
        in_specs=[
            pl.BlockSpec((NC, RB, 16), lambda i: (0, i, 0)),
            pl.BlockSpec((RB, F), lambda i: (i, 0)),
            pl.BlockSpec((F, F), lambda i: (0, 0)),
            pl.BlockSpec((F, F), lambda i: (0, 0)),
            pl.BlockSpec((F, C), lambda i: (0, 0)),
        ],
        out_specs=[
            pl.BlockSpec((RB, FH), lambda i: (i, 0)),
            pl.BlockSpec((RB, FH), lambda i: (i, 0)),
            pl.BlockSpec((RB, FH), lambda i: (i, 0)),
            pl.BlockSpec((RB, FH), lambda i: (i, 0)),
            pl.BlockSpec((RB, C), lambda i: (i, 0)),
            pl.BlockSpec((RB, 1), lambda i: (i, 0)),
            pl.BlockSpec((RB, 1), lambda i: (i, 0)),
        ],
        out_shape=[
            jax.ShapeDtypeStruct((N, FH), jnp.float32),
            jax.ShapeDtypeStruct((N, FH), jnp.float32),
            jax.ShapeDtypeStruct((N, FH), jnp.float32),
            jax.ShapeDtypeStruct((N, FH), jnp.float32),
            jax.ShapeDtypeStruct((N, C), jnp.float32),
            jax.ShapeDtypeStruct((N, 1), jnp.float32),
            jax.ShapeDtypeStruct((N, 1), jnp.float32),
        ],
    )(counts, x, W0, R0, R1)


def _mid_body(s0_ref, s1_ref, s2_ref, s3_ref, dii_ref, dio_ref, b0_ref,
              w1_ref, z1_ref, cat_ref):
    sg = jnp.concatenate([s0_ref[...], s1_ref[...]], axis=1)
    sz = jnp.concatenate([s2_ref[...], s3_ref[...]], axis=1)
    x1 = jnp.maximum(sg * dii_ref[...] + b0_ref[...] + sz, 0.0)
    h1 = jnp.dot(x1, w1_ref[...], preferred_element_type=jnp.float32) * dio_ref[...]
    cat_ref[...] = jnp.concatenate([h1, z1_ref[...]], axis=1)


def _mid_call(s0, s1, s2, s3, dii, dio, b0, W1, z1):
    return pl.pallas_call(
        _mid_body,
        grid=(N // RB,),
        in_specs=[
            pl.BlockSpec((RB, FH), lambda i: (i, 0)),
            pl.BlockSpec((RB, FH), lambda i: (i, 0)),
            pl.BlockSpec((RB, FH), lambda i: (i, 0)),
            pl.BlockSpec((RB, FH), lambda i: (i, 0)),
            pl.BlockSpec((RB, 1), lambda i: (i, 0)),
            pl.BlockSpec((RB, 1), lambda i: (i, 0)),
            pl.BlockSpec((1, F), lambda i: (0, 0)),
            pl.BlockSpec((F, C), lambda i: (0, 0)),
            pl.BlockSpec((RB, C), lambda i: (i, 0)),
        ],
        out_specs=pl.BlockSpec((RB, D1), lambda i: (i, 0)),
        out_shape=jax.ShapeDtypeStruct((N, D1), jnp.float32),
    )(s0, s1, s2, s3, dii, dio, b0, W1, z1)


def _final_body(p_ref, dii_ref, b1_ref, y_ref):
    p = p_ref[...]
    s_h = p[0, :, 0, :] + p[1, :, 0, :]
    s_z = p[0, :, 1, :] + p[1, :, 1, :]
    y_ref[...] = s_h * dii_ref[...] + b1_ref[...] + s_z


def _final_call(parts4, dii, b1):
    return pl.pallas_call(
        _final_body,
        grid=(N // RB,),
        in_specs=[
            pl.BlockSpec((NC, RB, 2, C), lambda i: (0, i, 0, 0)),
            pl.BlockSpec((RB, 1), lambda i: (i, 0)),
            pl.BlockSpec((1, C), lambda i: (0, 0)),
        ],
        out_specs=pl.BlockSpec((RB, C), lambda i: (i, 0)),
        out_shape=jax.ShapeDtypeStruct((N, C), jnp.float32),
    )(parts4, dii, b1)


def kernel(raw_x, edge_index, W0, b0, W1, b1, R0, R1):
    src = edge_index[0].astype(jnp.int32)
    dst = edge_index[1].astype(jnp.int32)
    src16 = src.reshape(NS, (E // B) // NS, B)
    dst16 = dst.reshape(NS, (E // B) // NS, B)
    src32 = src.reshape(NC * NS, (E // B) // (NC * NS), B)
    dst32 = dst.reshape(NC * NS, (E // B) // (NC * NS), B)

    counts = _deg_call(src16, dst16, jnp.zeros((N, 16), jnp.float32))
    f0, f1, f2, f3, z1, dio, dii = _pre_call(counts, raw_x, W0, R0, R1)
    s0, s1, s2, s3 = _spmm0_call(f0, f1, f2, f3, src16, dst16,
                                 jnp.zeros((N, FH), jnp.float32))
    cat1 = _mid_call(s0, s1, s2, s3, dii, dio, b0.reshape(1, F), W1, z1)
    parts = _spmm1_call(cat1, src32, dst32, jnp.zeros((N, D1), jnp.float32))
    y = _final_call(parts.reshape(NC, N, 2, C), dii, b1.reshape(1, C))
    return y


# speedup vs baseline: 9.7440x; 1.3115x over previous
import functools

import jax
import jax.numpy as jnp
from jax import lax
from jax.experimental import pallas as pl
from jax.experimental.pallas import tpu as pltpu
from jax.experimental.pallas import tpu_sc as plsc

N = 10000
E = 320000
F = 128
C = 40
NC = 2
NS = 16
B = 80
RB = 1000
D1 = 2 * C

CPT = 640


def _tile_row_base(s):
    return jnp.minimum(s * CPT, N - CPT)

_mesh = plsc.VectorSubcoreMesh(core_axis_name="c", subcore_axis_name="s")
_sc_params = pltpu.CompilerParams(use_tc_tiling_on_sc=False)


def _deg_body(src_hbm, dst_hbm, z16_hbm, out_hbm, acc, idx_v, ones_v):
    c = lax.axis_index("c")
    s = lax.axis_index("s")
    r0 = _tile_row_base(s)

    def fill_ones(i, _):
        ones_v[i, :] = jnp.ones((16,), jnp.float32)
        return 0

    lax.fori_loop(0, B, fill_ones, 0)
    pltpu.sync_copy(z16_hbm.at[pl.ds(r0, CPT)], acc.at[pl.ds(r0, CPT)])
    plsc.subcore_barrier()

    rows_per_tile = (E // B) // NS

    def run(idx_hbm):
        pltpu.sync_copy(idx_hbm.at[s], idx_v)

        def chunk(k, _):
            pltpu.sync_copy(ones_v, acc.at[idx_v.at[k]], add=True)
            return 0

        lax.fori_loop(0, rows_per_tile, chunk, 0)

    @pl.when(c == 0)
    def _():
        run(src_hbm)

    @pl.when(c == 1)
    def _():
        run(dst_hbm)

    plsc.subcore_barrier()
    pltpu.sync_copy(acc.at[pl.ds(r0, CPT)], out_hbm.at[c, pl.ds(r0, CPT)])


_deg_call = pl.kernel(
    _deg_body,
    out_type=jax.ShapeDtypeStruct((NC, N, 16), jnp.float32),
    mesh=_mesh,
    compiler_params=_sc_params,
    scratch_types=[
        pltpu.VMEM_SHARED((N, 16), jnp.float32),
        pltpu.VMEM(((E // B) // NS, B), jnp.int32),
        pltpu.VMEM((B, 16), jnp.float32),
    ],
)


FH = F // 2


NBUF = 4


def _spmm_pipeline(feat_hbm, acc, sidx, didx, bufs, sems, nchunks):
    for j in range(NBUF):
        pltpu.async_copy(feat_hbm.at[sidx.at[j]], bufs[j], sems[j])

    def body(p, _):
        for j in range(NBUF):
            k = NBUF * p + j
            pltpu.make_async_copy(feat_hbm.at[sidx.at[k]], bufs[j], sems[j]).wait()
            pltpu.sync_copy(bufs[j], acc.at[didx.at[k]], add=True)

            @pl.when(k + NBUF < nchunks)
            def _():
                pltpu.async_copy(feat_hbm.at[sidx.at[k + NBUF]], bufs[j], sems[j])

        return 0

    lax.fori_loop(0, nchunks // NBUF, body, 0)
    for j in range(nchunks % NBUF):
        k = (nchunks // NBUF) * NBUF + j
        pltpu.make_async_copy(feat_hbm.at[sidx.at[k]], bufs[j], sems[j]).wait()
        pltpu.sync_copy(bufs[j], acc.at[didx.at[k]], add=True)


def _spmm0_body(f0_hbm, f1_hbm, f2_hbm, f3_hbm, src_hbm, dst_hbm, zf_hbm,
                o0_hbm, o1_hbm, o2_hbm, o3_hbm,
                acc, sidx, didx, b0_, b1_, b2_, b3_, g0, g1, g2, g3):
    c = lax.axis_index("c")
    s = lax.axis_index("s")
    r0 = _tile_row_base(s)
    rows_per_tile = (E // B) // NS
    pltpu.sync_copy(src_hbm.at[s], sidx)
    pltpu.sync_copy(dst_hbm.at[s], didx)
    bufs = (b0_, b1_, b2_, b3_)
    sems = (g0, g1, g2, g3)

    def one_pass(feat_hbm, out_hbm):
        pltpu.sync_copy(zf_hbm.at[pl.ds(r0, CPT)], acc.at[pl.ds(r0, CPT)])
        plsc.subcore_barrier()
        _spmm_pipeline(feat_hbm, acc, sidx, didx, bufs, sems, rows_per_tile)
        plsc.subcore_barrier()
        pltpu.sync_copy(acc.at[pl.ds(r0, CPT)], out_hbm.at[pl.ds(r0, CPT)])
        plsc.subcore_barrier()

    @pl.when(c == 0)
    def _():
        one_pass(f0_hbm, o0_hbm)
        one_pass(f1_hbm, o1_hbm)

    @pl.when(c == 1)
    def _():
        one_pass(f2_hbm, o2_hbm)
        one_pass(f3_hbm, o3_hbm)


_spmm0_call = pl.kernel(
    _spmm0_body,
    out_type=tuple(jax.ShapeDtypeStruct((N, FH), jnp.float32) for _ in range(4)),
    mesh=_mesh,
    compiler_params=_sc_params,
    scratch_types=[
        pltpu.VMEM_SHARED((N, FH), jnp.float32),
        pltpu.VMEM(((E // B) // NS, B), jnp.int32),
        pltpu.VMEM(((E // B) // NS, B), jnp.int32),
        pltpu.VMEM((B, FH), jnp.float32),
        pltpu.VMEM((B, FH), jnp.float32),
        pltpu.VMEM((B, FH), jnp.float32),
        pltpu.VMEM((B, FH), jnp.float32),
        pltpu.SemaphoreType.DMA,
        pltpu.SemaphoreType.DMA,
        pltpu.SemaphoreType.DMA,
        pltpu.SemaphoreType.DMA,
    ],
)


def _spmm1_body(cat_hbm, src_hbm, dst_hbm, zd_hbm, out_hbm,
                acc, sidx, didx, b0_, b1_, b2_, b3_, g0, g1, g2, g3):
    c = lax.axis_index("c")
    s = lax.axis_index("s")
    r0 = _tile_row_base(s)
    pltpu.sync_copy(zd_hbm.at[pl.ds(r0, CPT)], acc.at[pl.ds(r0, CPT)])
    plsc.subcore_barrier()

    rows_per_worker = (E // B) // (NC * NS)
    wid = c * NS + s
    pltpu.sync_copy(src_hbm.at[wid], sidx)
    pltpu.sync_copy(dst_hbm.at[wid], didx)

    _spmm_pipeline(cat_hbm, acc, sidx, didx, (b0_, b1_, b2_, b3_),
                   (g0, g1, g2, g3), rows_per_worker)
    plsc.subcore_barrier()
    pltpu.sync_copy(acc.at[pl.ds(r0, CPT)], out_hbm.at[c, pl.ds(r0, CPT)])


_spmm1_call = pl.kernel(
    _spmm1_body,
    out_type=jax.ShapeDtypeStruct((NC, N, D1), jnp.float32),
    mesh=_mesh,
    compiler_params=_sc_params,
    scratch_types=[
        pltpu.VMEM_SHARED((N, D1), jnp.float32),
        pltpu.VMEM(((E // B) // (NC * NS), B), jnp.int32),
        pltpu.VMEM(((E // B) // (NC * NS), B), jnp.int32),
        pltpu.VMEM((B, D1), jnp.float32),
        pltpu.VMEM((B, D1), jnp.float32),
        pltpu.VMEM((B, D1), jnp.float32),
        pltpu.VMEM((B, D1), jnp.float32),
        pltpu.SemaphoreType.DMA,
        pltpu.SemaphoreType.DMA,
        pltpu.SemaphoreType.DMA,
        pltpu.SemaphoreType.DMA,
    ],
)


def _pre_body(cnt_ref, x_ref, w0_ref, r0_ref, r1_ref,
              f0_ref, f1_ref, f2_ref, f3_ref, z1_ref, dio_ref, dii_ref):
    cnt = cnt_ref[...]
    dout = lax.rsqrt(jnp.maximum(jnp.max(cnt[0], axis=1, keepdims=True), 1.0))
    din = lax.rsqrt(jnp.maximum(jnp.max(cnt[1], axis=1, keepdims=True), 1.0))
    x = x_ref[...]
    z0 = jnp.dot(x, r0_ref[...], preferred_element_type=jnp.float32)
    fg = jnp.dot(x, w0_ref[...], preferred_element_type=jnp.float32) * dout
    f0_ref[...] = fg[:, :FH]
    f1_ref[...] = fg[:, FH:]
    f2_ref[...] = z0[:, :FH]
    f3_ref[...] = z0[:, FH:]
    z1_ref[...] = jnp.dot(z0, r1_ref[...], preferred_element_type=jnp.float32)
    dio_ref[...] = dout
    dii_ref[...] = din


def _pre_call(counts, x, W0, R0, R1):
    return pl.pallas_call(
        _pre_body,
        grid=(N // RB,),
        in_specs=[
            pl.BlockSpec((NC, RB, 16), lambda i: (0, i, 0)),
            pl.BlockSpec((RB, F), lambda i: (i, 0)),
            pl.BlockSpec((F, F), lambda i: (0, 0)),
            pl.BlockSpec((F, F), lambda i: (0, 0)),
            pl.BlockSpec((F, C), lambda i: (0, 0)),
        ],
        out_specs=[
            pl.BlockSpec((RB, FH), lambda i: (i, 0)),
            pl.BlockSpec((RB, FH), lambda i: (i, 0)),
            pl.BlockSpec((RB, FH), lambda i: (i, 0)),
            pl.BlockSpec((RB, FH), lambda i: (i, 0)),
            pl.BlockSpec((RB, C), lambda i: (i, 0)),
            pl.BlockSpec((RB, 1), lambda i: (i, 0)),
            pl.BlockSpec((RB, 1), lambda i: (i, 0)),
        ],
        out_shape=[
            jax.ShapeDtypeStruct((N, FH), jnp.float32),
            jax.ShapeDtypeStruct((N, FH), jnp.float32),
            jax.ShapeDtypeStruct((N, FH), jnp.float32),
            jax.ShapeDtypeStruct((N, FH), jnp.float32),
            jax.ShapeDtypeStruct((N, C), jnp.float32),
            jax.ShapeDtypeStruct((N, 1), jnp.float32),
            jax.ShapeDtypeStruct((N, 1), jnp.float32),
        ],
    )(counts, x, W0, R0, R1)


def _mid_body(s0_ref, s1_ref, s2_ref, s3_ref, dii_ref, dio_ref, b0_ref,
              w1_ref, z1_ref, cat_ref):
    sg = jnp.concatenate([s0_ref[...], s1_ref[...]], axis=1)
    sz = jnp.concatenate([s2_ref[...], s3_ref[...]], axis=1)
    x1 = jnp.maximum(sg * dii_ref[...] + b0_ref[...] + sz, 0.0)
    h1 = jnp.dot(x1, w1_ref[...], preferred_element_type=jnp.float32) * dio_ref[...]
    cat_ref[...] = jnp.concatenate([h1, z1_ref[...]], axis=1)


def _mid_call(s0, s1, s2, s3, dii, dio, b0, W1, z1):
    return pl.pallas_call(
        _mid_body,
        grid=(N // RB,),
        in_specs=[
            pl.BlockSpec((RB, FH), lambda i: (i, 0)),
            pl.BlockSpec((RB, FH), lambda i: (i, 0)),
            pl.BlockSpec((RB, FH), lambda i: (i, 0)),
            pl.BlockSpec((RB, FH), lambda i: (i, 0)),
            pl.BlockSpec((RB, 1), lambda i: (i, 0)),
            pl.BlockSpec((RB, 1), lambda i: (i, 0)),
            pl.BlockSpec((1, F), lambda i: (0, 0)),
            pl.BlockSpec((F, C), lambda i: (0, 0)),
            pl.BlockSpec((RB, C), lambda i: (i, 0)),
        ],
        out_specs=pl.BlockSpec((RB, D1), lambda i: (i, 0)),
        out_shape=jax.ShapeDtypeStruct((N, D1), jnp.float32),
    )(s0, s1, s2, s3, dii, dio, b0, W1, z1)


def _final_body(p_ref, dii_ref, b1_ref, y_ref):
    p = p_ref[...]
    s_h = p[0, :, 0, :] + p[1, :, 0, :]
    s_z = p[0, :, 1, :] + p[1, :, 1, :]
    y_ref[...] = s_h * dii_ref[...] + b1_ref[...] + s_z


def _final_call(parts4, dii, b1):
    return pl.pallas_call(
        _final_body,
        grid=(N // RB,),
        in_specs=[
            pl.BlockSpec((NC, RB, 2, C), lambda i: (0, i, 0, 0)),
            pl.BlockSpec((RB, 1), lambda i: (i, 0)),
            pl.BlockSpec((1, C), lambda i: (0, 0)),
        ],
        out_specs=pl.BlockSpec((RB, C), lambda i: (i, 0)),
        out_shape=jax.ShapeDtypeStruct((N, C), jnp.float32),
    )(parts4, dii, b1)


def kernel(raw_x, edge_index, W0, b0, W1, b1, R0, R1):
    src = edge_index[0].astype(jnp.int32)
    dst = edge_index[1].astype(jnp.int32)
    src16 = src.reshape(NS, (E // B) // NS, B)
    dst16 = dst.reshape(NS, (E // B) // NS, B)
    src32 = src.reshape(NC * NS, (E // B) // (NC * NS), B)
    dst32 = dst.reshape(NC * NS, (E // B) // (NC * NS), B)

    counts = _deg_call(src16, dst16, jnp.zeros((N, 16), jnp.float32))
    f0, f1, f2, f3, z1, dio, dii = _pre_call(counts, raw_x, W0, R0, R1)
    s0, s1, s2, s3 = _spmm0_call(f0, f1, f2, f3, src16, dst16,
                                 jnp.zeros((N, FH), jnp.float32))
    cat1 = _mid_call(s0, s1, s2, s3, dii, dio, b0.reshape(1, F), W1, z1)
    parts = _spmm1_call(cat1, src32, dst32, jnp.zeros((N, D1), jnp.float32))
    y = _final_call(parts.reshape(NC, N, 2, C), dii, b1.reshape(1, C))
    return y


# speedup vs baseline: 10.3184x; 1.0589x over previous
import functools

import jax
import jax.numpy as jnp
from jax import lax
from jax.experimental import pallas as pl
from jax.experimental.pallas import tpu as pltpu
from jax.experimental.pallas import tpu_sc as plsc

N = 10000
E = 320000
F = 128
C = 40
NC = 2
NS = 16
B = 80
RB = 1000
D1 = 2 * C

CPT = 640


def _tile_row_base(s):
    return jnp.minimum(s * CPT, N - CPT)

_mesh = plsc.VectorSubcoreMesh(core_axis_name="c", subcore_axis_name="s")
_sc_params = pltpu.CompilerParams(use_tc_tiling_on_sc=False)


NBUF = 4


def _deg_body(src_hbm, dst_hbm, z16_hbm, out_hbm, acc, idx_v, ones_v,
              m0, m1, m2, m3):
    c = lax.axis_index("c")
    s = lax.axis_index("s")
    r0 = _tile_row_base(s)
    dsems = (m0, m1, m2, m3)

    def fill_ones(i, _):
        ones_v[i, :] = jnp.ones((16,), jnp.float32)
        return 0

    lax.fori_loop(0, B, fill_ones, 0)
    pltpu.sync_copy(z16_hbm.at[pl.ds(r0, CPT)], acc.at[pl.ds(r0, CPT)])
    plsc.subcore_barrier()

    rows_per_tile = (E // B) // NS

    def run(idx_hbm, sems):
        pltpu.sync_copy(idx_hbm.at[s], idx_v)
        for j in range(NBUF):
            pltpu.async_copy(ones_v, acc.at[idx_v.at[j]], sems[j], add=True)

        def group(p, _):
            for j in range(NBUF):
                k = NBUF * p + j
                pltpu.make_async_copy(ones_v, acc.at[idx_v.at[k]], sems[j]).wait()

                @pl.when(k + NBUF < rows_per_tile)
                def _():
                    pltpu.async_copy(ones_v, acc.at[idx_v.at[k + NBUF]],
                                     sems[j], add=True)

            return 0

        lax.fori_loop(0, rows_per_tile // NBUF, group, 0)
        for j in range(rows_per_tile % NBUF):
            k = (rows_per_tile // NBUF) * NBUF + j
            pltpu.make_async_copy(ones_v, acc.at[idx_v.at[k]], sems[j]).wait()

    @pl.when(c == 0)
    def _():
        run(src_hbm, dsems)

    @pl.when(c == 1)
    def _():
        run(dst_hbm, dsems)

    plsc.subcore_barrier()
    pltpu.sync_copy(acc.at[pl.ds(r0, CPT)], out_hbm.at[c, pl.ds(r0, CPT)])


_deg_call = pl.kernel(
    _deg_body,
    out_type=jax.ShapeDtypeStruct((NC, N, 16), jnp.float32),
    mesh=_mesh,
    compiler_params=_sc_params,
    scratch_types=[
        pltpu.VMEM_SHARED((N, 16), jnp.float32),
        pltpu.VMEM(((E // B) // NS, B), jnp.int32),
        pltpu.VMEM((B, 16), jnp.float32),
        pltpu.SemaphoreType.DMA,
        pltpu.SemaphoreType.DMA,
        pltpu.SemaphoreType.DMA,
        pltpu.SemaphoreType.DMA,
    ],
)


FH = F // 2


def _spmm_pipeline(feat_hbm, acc, sidx, didx, bufs, sems, nchunks):
    for j in range(NBUF):
        pltpu.async_copy(feat_hbm.at[sidx.at[j]], bufs[j], sems[j])

    def body(p, _):
        for j in range(NBUF):
            k = NBUF * p + j
            pltpu.make_async_copy(feat_hbm.at[sidx.at[k]], bufs[j], sems[j]).wait()
            pltpu.sync_copy(bufs[j], acc.at[didx.at[k]], add=True)

            @pl.when(k + NBUF < nchunks)
            def _():
                pltpu.async_copy(feat_hbm.at[sidx.at[k + NBUF]], bufs[j], sems[j])

        return 0

    lax.fori_loop(0, nchunks // NBUF, body, 0)
    for j in range(nchunks % NBUF):
        k = (nchunks // NBUF) * NBUF + j
        pltpu.make_async_copy(feat_hbm.at[sidx.at[k]], bufs[j], sems[j]).wait()
        pltpu.sync_copy(bufs[j], acc.at[didx.at[k]], add=True)


def _spmm0_body(f0_hbm, f1_hbm, f2_hbm, f3_hbm, src_hbm, dst_hbm, zf_hbm,
                o0_hbm, o1_hbm, o2_hbm, o3_hbm,
                acc, sidx, didx, b0_, b1_, b2_, b3_, g0, g1, g2, g3):
    c = lax.axis_index("c")
    s = lax.axis_index("s")
    r0 = _tile_row_base(s)
    rows_per_tile = (E // B) // NS
    pltpu.sync_copy(src_hbm.at[s], sidx)
    pltpu.sync_copy(dst_hbm.at[s], didx)
    bufs = (b0_, b1_, b2_, b3_)
    sems = (g0, g1, g2, g3)

    def one_pass(feat_hbm, out_hbm):
        pltpu.sync_copy(zf_hbm.at[pl.ds(r0, CPT)], acc.at[pl.ds(r0, CPT)])
        plsc.subcore_barrier()
        _spmm_pipeline(feat_hbm, acc, sidx, didx, bufs, sems, rows_per_tile)
        plsc.subcore_barrier()
        pltpu.sync_copy(acc.at[pl.ds(r0, CPT)], out_hbm.at[pl.ds(r0, CPT)])
        plsc.subcore_barrier()

    @pl.when(c == 0)
    def _():
        one_pass(f0_hbm, o0_hbm)
        one_pass(f1_hbm, o1_hbm)

    @pl.when(c == 1)
    def _():
        one_pass(f2_hbm, o2_hbm)
        one_pass(f3_hbm, o3_hbm)


_spmm0_call = pl.kernel(
    _spmm0_body,
    out_type=tuple(jax.ShapeDtypeStruct((N, FH), jnp.float32) for _ in range(4)),
    mesh=_mesh,
    compiler_params=_sc_params,
    scratch_types=[
        pltpu.VMEM_SHARED((N, FH), jnp.float32),
        pltpu.VMEM(((E // B) // NS, B), jnp.int32),
        pltpu.VMEM(((E // B) // NS, B), jnp.int32),
        pltpu.VMEM((B, FH), jnp.float32),
        pltpu.VMEM((B, FH), jnp.float32),
        pltpu.VMEM((B, FH), jnp.float32),
        pltpu.VMEM((B, FH), jnp.float32),
        pltpu.SemaphoreType.DMA,
        pltpu.SemaphoreType.DMA,
        pltpu.SemaphoreType.DMA,
        pltpu.SemaphoreType.DMA,
    ],
)


def _spmm1_body(cat_hbm, src_hbm, dst_hbm, zd_hbm, out_hbm,
                acc, sidx, didx, b0_, b1_, b2_, b3_, g0, g1, g2, g3):
    c = lax.axis_index("c")
    s = lax.axis_index("s")
    r0 = _tile_row_base(s)
    pltpu.sync_copy(zd_hbm.at[pl.ds(r0, CPT)], acc.at[pl.ds(r0, CPT)])
    plsc.subcore_barrier()

    rows_per_worker = (E // B) // (NC * NS)
    wid = c * NS + s
    pltpu.sync_copy(src_hbm.at[wid], sidx)
    pltpu.sync_copy(dst_hbm.at[wid], didx)

    _spmm_pipeline(cat_hbm, acc, sidx, didx, (b0_, b1_, b2_, b3_),
                   (g0, g1, g2, g3), rows_per_worker)
    plsc.subcore_barrier()
    pltpu.sync_copy(acc.at[pl.ds(r0, CPT)], out_hbm.at[c, pl.ds(r0, CPT)])


_spmm1_call = pl.kernel(
    _spmm1_body,
    out_type=jax.ShapeDtypeStruct((NC, N, D1), jnp.float32),
    mesh=_mesh,
    compiler_params=_sc_params,
    scratch_types=[
        pltpu.VMEM_SHARED((N, D1), jnp.float32),
        pltpu.VMEM(((E // B) // (NC * NS), B), jnp.int32),
        pltpu.VMEM(((E // B) // (NC * NS), B), jnp.int32),
        pltpu.VMEM((B, D1), jnp.float32),
        pltpu.VMEM((B, D1), jnp.float32),
        pltpu.VMEM((B, D1), jnp.float32),
        pltpu.VMEM((B, D1), jnp.float32),
        pltpu.SemaphoreType.DMA,
        pltpu.SemaphoreType.DMA,
        pltpu.SemaphoreType.DMA,
        pltpu.SemaphoreType.DMA,
    ],
)


def _bf16_dot(a, b):
    return jnp.dot(a.astype(jnp.bfloat16), b.astype(jnp.bfloat16),
                   preferred_element_type=jnp.float32)


def _pre_a_body(x_ref, w0_ref, r0_ref, r1_ref,
                u_ref, f2_ref, f3_ref, z1_ref):
    x = x_ref[...]
    z0 = _bf16_dot(x, r0_ref[...])
    u_ref[...] = _bf16_dot(x, w0_ref[...])
    f2_ref[...] = z0[:, :FH]
    f3_ref[...] = z0[:, FH:]
    z1_ref[...] = _bf16_dot(z0, r1_ref[...])


def _pre_a_call(x, W0, R0, R1):
    return pl.pallas_call(
        _pre_a_body,
        grid=(N // RB,),
        in_specs=[
            pl.BlockSpec((RB, F), lambda i: (i, 0)),
            pl.BlockSpec((F, F), lambda i: (0, 0)),
            pl.BlockSpec((F, F), lambda i: (0, 0)),
            pl.BlockSpec((F, C), lambda i: (0, 0)),
        ],
        out_specs=[
            pl.BlockSpec((RB, F), lambda i: (i, 0)),
            pl.BlockSpec((RB, FH), lambda i: (i, 0)),
            pl.BlockSpec((RB, FH), lambda i: (i, 0)),
            pl.BlockSpec((RB, C), lambda i: (i, 0)),
        ],
        out_shape=[
            jax.ShapeDtypeStruct((N, F), jnp.float32),
            jax.ShapeDtypeStruct((N, FH), jnp.float32),
            jax.ShapeDtypeStruct((N, FH), jnp.float32),
            jax.ShapeDtypeStruct((N, C), jnp.float32),
        ],
    )(x, W0, R0, R1)


def _pre_b_body(cnt_ref, u_ref, f0_ref, f1_ref, dio_ref, dii_ref):
    cnt = cnt_ref[...]
    dout = lax.rsqrt(jnp.maximum(jnp.max(cnt[0], axis=1, keepdims=True), 1.0))
    din = lax.rsqrt(jnp.maximum(jnp.max(cnt[1], axis=1, keepdims=True), 1.0))
    fg = u_ref[...] * dout
    f0_ref[...] = fg[:, :FH]
    f1_ref[...] = fg[:, FH:]
    dio_ref[...] = dout
    dii_ref[...] = din


def _pre_b_call(counts, u):
    return pl.pallas_call(
        _pre_b_body,
        grid=(N // RB,),
        in_specs=[
            pl.BlockSpec((NC, RB, 16), lambda i: (0, i, 0)),
            pl.BlockSpec((RB, F), lambda i: (i, 0)),
        ],
        out_specs=[
            pl.BlockSpec((RB, FH), lambda i: (i, 0)),
            pl.BlockSpec((RB, FH), lambda i: (i, 0)),
            pl.BlockSpec((RB, 1), lambda i: (i, 0)),
            pl.BlockSpec((RB, 1), lambda i: (i, 0)),
        ],
        out_shape=[
            jax.ShapeDtypeStruct((N, FH), jnp.float32),
            jax.ShapeDtypeStruct((N, FH), jnp.float32),
            jax.ShapeDtypeStruct((N, 1), jnp.float32),
            jax.ShapeDtypeStruct((N, 1), jnp.float32),
        ],
    )(counts, u)


def _mid_body(s0_ref, s1_ref, s2_ref, s3_ref, dii_ref, dio_ref, b0_ref,
              w1_ref, z1_ref, cat_ref):
    sg = jnp.concatenate([s0_ref[...], s1_ref[...]], axis=1)
    sz = jnp.concatenate([s2_ref[...], s3_ref[...]], axis=1)
    x1 = jnp.maximum(sg * dii_ref[...] + b0_ref[...] + sz, 0.0)
    h1 = _bf16_dot(x1, w1_ref[...]) * dio_ref[...]
    cat_ref[...] = jnp.concatenate([h1, z1_ref[...]], axis=1)


def _mid_call(s0, s1, s2, s3, dii, dio, b0, W1, z1):
    return pl.pallas_call(
        _mid_body,
        grid=(N // RB,),
        in_specs=[
            pl.BlockSpec((RB, FH), lambda i: (i, 0)),
            pl.BlockSpec((RB, FH), lambda i: (i, 0)),
            pl.BlockSpec((RB, FH), lambda i: (i, 0)),
            pl.BlockSpec((RB, FH), lambda i: (i, 0)),
            pl.BlockSpec((RB, 1), lambda i: (i, 0)),
            pl.BlockSpec((RB, 1), lambda i: (i, 0)),
            pl.BlockSpec((1, F), lambda i: (0, 0)),
            pl.BlockSpec((F, C), lambda i: (0, 0)),
            pl.BlockSpec((RB, C), lambda i: (i, 0)),
        ],
        out_specs=pl.BlockSpec((RB, D1), lambda i: (i, 0)),
        out_shape=jax.ShapeDtypeStruct((N, D1), jnp.float32),
    )(s0, s1, s2, s3, dii, dio, b0, W1, z1)


def _final_body(p_ref, dii_ref, b1_ref, y_ref):
    p = p_ref[...]
    s_h = p[0, :, 0, :] + p[1, :, 0, :]
    s_z = p[0, :, 1, :] + p[1, :, 1, :]
    y_ref[...] = s_h * dii_ref[...] + b1_ref[...] + s_z


def _final_call(parts4, dii, b1):
    return pl.pallas_call(
        _final_body,
        grid=(N // RB,),
        in_specs=[
            pl.BlockSpec((NC, RB, 2, C), lambda i: (0, i, 0, 0)),
            pl.BlockSpec((RB, 1), lambda i: (i, 0)),
            pl.BlockSpec((1, C), lambda i: (0, 0)),
        ],
        out_specs=pl.BlockSpec((RB, C), lambda i: (i, 0)),
        out_shape=jax.ShapeDtypeStruct((N, C), jnp.float32),
    )(parts4, dii, b1)


def kernel(raw_x, edge_index, W0, b0, W1, b1, R0, R1):
    src = edge_index[0].astype(jnp.int32)
    dst = edge_index[1].astype(jnp.int32)
    src16 = src.reshape(NS, (E // B) // NS, B)
    dst16 = dst.reshape(NS, (E // B) // NS, B)
    src32 = src.reshape(NC * NS, (E // B) // (NC * NS), B)
    dst32 = dst.reshape(NC * NS, (E // B) // (NC * NS), B)

    counts = _deg_call(src16, dst16, jnp.zeros((N, 16), jnp.float32))
    u, f2, f3, z1 = _pre_a_call(raw_x, W0, R0, R1)
    f0, f1, dio, dii = _pre_b_call(counts, u)
    s0, s1, s2, s3 = _spmm0_call(f0, f1, f2, f3, src16, dst16,
                                 jnp.zeros((N, FH), jnp.float32))
    cat1 = _mid_call(s0, s1, s2, s3, dii, dio, b0.reshape(1, F), W1, z1)
    parts = _spmm1_call(cat1, src32, dst32, jnp.zeros((N, D1), jnp.float32))
    y = _final_call(parts.reshape(NC, N, 2, C), dii, b1.reshape(1, C))
    return y


# speedup vs baseline: 11.3851x; 1.1034x over previous
import functools

import jax
import jax.numpy as jnp
from jax import lax
from jax.experimental import pallas as pl
from jax.experimental.pallas import tpu as pltpu
from jax.experimental.pallas import tpu_sc as plsc

N = 10000
E = 320000
F = 128
C = 40
NC = 2
NS = 16
B = 80
RB = 1000
D1 = 2 * C

CPT = 640


def _tile_row_base(s):
    return jnp.minimum(s * CPT, N - CPT)

_mesh = plsc.VectorSubcoreMesh(core_axis_name="c", subcore_axis_name="s")
_sc_params = pltpu.CompilerParams(use_tc_tiling_on_sc=False)


NBUF = 4


def _deg_body(src_hbm, dst_hbm, z16_hbm, out_hbm, acc, idx_v, ones_v,
              m0, m1, m2, m3):
    c = lax.axis_index("c")
    s = lax.axis_index("s")
    r0 = _tile_row_base(s)
    dsems = (m0, m1, m2, m3)

    def fill_ones(i, _):
        ones_v[i, :] = jnp.ones((16,), jnp.float32)
        return 0

    lax.fori_loop(0, B, fill_ones, 0)
    pltpu.sync_copy(z16_hbm.at[pl.ds(r0, CPT)], acc.at[pl.ds(r0, CPT)])
    plsc.subcore_barrier()

    rows_per_tile = (E // B) // NS

    def run(idx_hbm, sems):
        half = (E // B) // (NC * NS)
        pltpu.sync_copy(idx_hbm.at[2 * s], idx_v.at[pl.ds(0, half)])
        pltpu.sync_copy(idx_hbm.at[2 * s + 1], idx_v.at[pl.ds(half, half)])
        for j in range(NBUF):
            pltpu.async_copy(ones_v, acc.at[idx_v.at[j]], sems[j], add=True)

        def group(p, _):
            for j in range(NBUF):
                k = NBUF * p + j
                pltpu.make_async_copy(ones_v, acc.at[idx_v.at[k]], sems[j]).wait()

                @pl.when(k + NBUF < rows_per_tile)
                def _():
                    pltpu.async_copy(ones_v, acc.at[idx_v.at[k + NBUF]],
                                     sems[j], add=True)

            return 0

        lax.fori_loop(0, rows_per_tile // NBUF, group, 0)
        for j in range(rows_per_tile % NBUF):
            k = (rows_per_tile // NBUF) * NBUF + j
            pltpu.make_async_copy(ones_v, acc.at[idx_v.at[k]], sems[j]).wait()

    @pl.when(c == 0)
    def _():
        run(src_hbm, dsems)

    @pl.when(c == 1)
    def _():
        run(dst_hbm, dsems)

    plsc.subcore_barrier()
    pltpu.sync_copy(acc.at[pl.ds(r0, CPT)], out_hbm.at[c, pl.ds(r0, CPT)])


_deg_call = pl.kernel(
    _deg_body,
    out_type=jax.ShapeDtypeStruct((NC, N, 16), jnp.float32),
    mesh=_mesh,
    compiler_params=_sc_params,
    scratch_types=[
        pltpu.VMEM_SHARED((N, 16), jnp.float32),
        pltpu.VMEM(((E // B) // NS, B), jnp.int32),
        pltpu.VMEM((B, 16), jnp.float32),
        pltpu.SemaphoreType.DMA,
        pltpu.SemaphoreType.DMA,
        pltpu.SemaphoreType.DMA,
        pltpu.SemaphoreType.DMA,
    ],
)


FH = F // 2


def _spmm_pipeline(feat_hbm, acc, sidx, didx, bufs, sems, nchunks):
    for j in range(NBUF):
        pltpu.async_copy(feat_hbm.at[sidx.at[j]], bufs[j], sems[j])

    def body(p, _):
        for j in range(NBUF):
            k = NBUF * p + j
            pltpu.make_async_copy(feat_hbm.at[sidx.at[k]], bufs[j], sems[j]).wait()
            pltpu.sync_copy(bufs[j], acc.at[didx.at[k]], add=True)

            @pl.when(k + NBUF < nchunks)
            def _():
                pltpu.async_copy(feat_hbm.at[sidx.at[k + NBUF]], bufs[j], sems[j])

        return 0

    lax.fori_loop(0, nchunks // NBUF, body, 0)
    for j in range(nchunks % NBUF):
        k = (nchunks // NBUF) * NBUF + j
        pltpu.make_async_copy(feat_hbm.at[sidx.at[k]], bufs[j], sems[j]).wait()
        pltpu.sync_copy(bufs[j], acc.at[didx.at[k]], add=True)


def _spmm0_body(f0_hbm, f1_hbm, f2_hbm, f3_hbm, src_hbm, dst_hbm, zf_hbm,
                o0_hbm, o1_hbm, o2_hbm, o3_hbm,
                acc, sidx, didx, b0_, b1_, b2_, b3_, g0, g1, g2, g3):
    c = lax.axis_index("c")
    s = lax.axis_index("s")
    r0 = _tile_row_base(s)
    rows_per_tile = (E // B) // NS
    half = (E // B) // (NC * NS)
    pltpu.sync_copy(src_hbm.at[2 * s], sidx.at[pl.ds(0, half)])
    pltpu.sync_copy(src_hbm.at[2 * s + 1], sidx.at[pl.ds(half, half)])
    pltpu.sync_copy(dst_hbm.at[2 * s], didx.at[pl.ds(0, half)])
    pltpu.sync_copy(dst_hbm.at[2 * s + 1], didx.at[pl.ds(half, half)])
    bufs = (b0_, b1_, b2_, b3_)
    sems = (g0, g1, g2, g3)

    def one_pass(feat_hbm, out_hbm):
        pltpu.sync_copy(zf_hbm.at[pl.ds(r0, CPT)], acc.at[pl.ds(r0, CPT)])
        plsc.subcore_barrier()
        _spmm_pipeline(feat_hbm, acc, sidx, didx, bufs, sems, rows_per_tile)
        plsc.subcore_barrier()
        pltpu.sync_copy(acc.at[pl.ds(r0, CPT)], out_hbm.at[pl.ds(r0, CPT)])
        plsc.subcore_barrier()

    @pl.when(c == 0)
    def _():
        one_pass(f0_hbm, o0_hbm)
        one_pass(f1_hbm, o1_hbm)

    @pl.when(c == 1)
    def _():
        one_pass(f2_hbm, o2_hbm)
        one_pass(f3_hbm, o3_hbm)


_spmm0_call = pl.kernel(
    _spmm0_body,
    out_type=tuple(jax.ShapeDtypeStruct((N, FH), jnp.float32) for _ in range(4)),
    mesh=_mesh,
    compiler_params=_sc_params,
    scratch_types=[
        pltpu.VMEM_SHARED((N, FH), jnp.float32),
        pltpu.VMEM(((E // B) // NS, B), jnp.int32),
        pltpu.VMEM(((E // B) // NS, B), jnp.int32),
        pltpu.VMEM((B, FH), jnp.float32),
        pltpu.VMEM((B, FH), jnp.float32),
        pltpu.VMEM((B, FH), jnp.float32),
        pltpu.VMEM((B, FH), jnp.float32),
        pltpu.SemaphoreType.DMA,
        pltpu.SemaphoreType.DMA,
        pltpu.SemaphoreType.DMA,
        pltpu.SemaphoreType.DMA,
    ],
)


def _spmm1_body(cat_hbm, src_hbm, dst_hbm, zd_hbm, out_hbm,
                acc, sidx, didx, b0_, b1_, b2_, b3_, g0, g1, g2, g3):
    c = lax.axis_index("c")
    s = lax.axis_index("s")
    r0 = _tile_row_base(s)
    pltpu.sync_copy(zd_hbm.at[pl.ds(r0, CPT)], acc.at[pl.ds(r0, CPT)])
    plsc.subcore_barrier()

    rows_per_worker = (E // B) // (NC * NS)
    wid = c * NS + s
    pltpu.sync_copy(src_hbm.at[wid], sidx)
    pltpu.sync_copy(dst_hbm.at[wid], didx)

    _spmm_pipeline(cat_hbm, acc, sidx, didx, (b0_, b1_, b2_, b3_),
                   (g0, g1, g2, g3), rows_per_worker)
    plsc.subcore_barrier()
    pltpu.sync_copy(acc.at[pl.ds(r0, CPT)], out_hbm.at[c, pl.ds(r0, CPT)])


_spmm1_call = pl.kernel(
    _spmm1_body,
    out_type=jax.ShapeDtypeStruct((NC, N, D1), jnp.float32),
    mesh=_mesh,
    compiler_params=_sc_params,
    scratch_types=[
        pltpu.VMEM_SHARED((N, D1), jnp.float32),
        pltpu.VMEM(((E // B) // (NC * NS), B), jnp.int32),
        pltpu.VMEM(((E // B) // (NC * NS), B), jnp.int32),
        pltpu.VMEM((B, D1), jnp.float32),
        pltpu.VMEM((B, D1), jnp.float32),
        pltpu.VMEM((B, D1), jnp.float32),
        pltpu.VMEM((B, D1), jnp.float32),
        pltpu.SemaphoreType.DMA,
        pltpu.SemaphoreType.DMA,
        pltpu.SemaphoreType.DMA,
        pltpu.SemaphoreType.DMA,
    ],
)


def _bf16_dot(a, b):
    return jnp.dot(a.astype(jnp.bfloat16), b.astype(jnp.bfloat16),
                   preferred_element_type=jnp.float32)


def _pre_a_body(x_ref, w0_ref, r0_ref, r1_ref,
                u_ref, f2_ref, f3_ref, z1_ref):
    x = x_ref[...]
    z0 = _bf16_dot(x, r0_ref[...])
    u_ref[...] = _bf16_dot(x, w0_ref[...])
    f2_ref[...] = z0[:, :FH]
    f3_ref[...] = z0[:, FH:]
    z1_ref[...] = _bf16_dot(z0, r1_ref[...])


def _pre_a_call(x, W0, R0, R1):
    return pl.pallas_call(
        _pre_a_body,
        grid=(N // RB,),
        in_specs=[
            pl.BlockSpec((RB, F), lambda i: (i, 0)),
            pl.BlockSpec((F, F), lambda i: (0, 0)),
            pl.BlockSpec((F, F), lambda i: (0, 0)),
            pl.BlockSpec((F, C), lambda i: (0, 0)),
        ],
        out_specs=[
            pl.BlockSpec((RB, F), lambda i: (i, 0)),
            pl.BlockSpec((RB, FH), lambda i: (i, 0)),
            pl.BlockSpec((RB, FH), lambda i: (i, 0)),
            pl.BlockSpec((RB, C), lambda i: (i, 0)),
        ],
        out_shape=[
            jax.ShapeDtypeStruct((N, F), jnp.float32),
            jax.ShapeDtypeStruct((N, FH), jnp.float32),
            jax.ShapeDtypeStruct((N, FH), jnp.float32),
            jax.ShapeDtypeStruct((N, C), jnp.float32),
        ],
    )(x, W0, R0, R1)


def _pre_b_body(cnt_ref, u_ref, f0_ref, f1_ref, dio_ref, dii_ref):
    cnt = cnt_ref[...]
    dout = lax.rsqrt(jnp.maximum(jnp.max(cnt[0], axis=1, keepdims=True), 1.0))
    din = lax.rsqrt(jnp.maximum(jnp.max(cnt[1], axis=1, keepdims=True), 1.0))
    fg = u_ref[...] * dout
    f0_ref[...] = fg[:, :FH]
    f1_ref[...] = fg[:, FH:]
    dio_ref[...] = dout
    dii_ref[...] = din


def _pre_b_call(counts, u):
    return pl.pallas_call(
        _pre_b_body,
        grid=(N // RB,),
        in_specs=[
            pl.BlockSpec((NC, RB, 16), lambda i: (0, i, 0)),
            pl.BlockSpec((RB, F), lambda i: (i, 0)),
        ],
        out_specs=[
            pl.BlockSpec((RB, FH), lambda i: (i, 0)),
            pl.BlockSpec((RB, FH), lambda i: (i, 0)),
            pl.BlockSpec((RB, 1), lambda i: (i, 0)),
            pl.BlockSpec((RB, 1), lambda i: (i, 0)),
        ],
        out_shape=[
            jax.ShapeDtypeStruct((N, FH), jnp.float32),
            jax.ShapeDtypeStruct((N, FH), jnp.float32),
            jax.ShapeDtypeStruct((N, 1), jnp.float32),
            jax.ShapeDtypeStruct((N, 1), jnp.float32),
        ],
    )(counts, u)


def _mid_body(s0_ref, s1_ref, s2_ref, s3_ref, dii_ref, dio_ref, b0_ref,
              w1_ref, z1_ref, cat_ref):
    sg = jnp.concatenate([s0_ref[...], s1_ref[...]], axis=1)
    sz = jnp.concatenate([s2_ref[...], s3_ref[...]], axis=1)
    x1 = jnp.maximum(sg * dii_ref[...] + b0_ref[...] + sz, 0.0)
    h1 = _bf16_dot(x1, w1_ref[...]) * dio_ref[...]
    cat_ref[...] = jnp.concatenate([h1, z1_ref[...]], axis=1)


def _mid_call(s0, s1, s2, s3, dii, dio, b0, W1, z1):
    return pl.pallas_call(
        _mid_body,
        grid=(N // RB,),
        in_specs=[
            pl.BlockSpec((RB, FH), lambda i: (i, 0)),
            pl.BlockSpec((RB, FH), lambda i: (i, 0)),
            pl.BlockSpec((RB, FH), lambda i: (i, 0)),
            pl.BlockSpec((RB, FH), lambda i: (i, 0)),
            pl.BlockSpec((RB, 1), lambda i: (i, 0)),
            pl.BlockSpec((RB, 1), lambda i: (i, 0)),
            pl.BlockSpec((1, F), lambda i: (0, 0)),
            pl.BlockSpec((F, C), lambda i: (0, 0)),
            pl.BlockSpec((RB, C), lambda i: (i, 0)),
        ],
        out_specs=pl.BlockSpec((RB, D1), lambda i: (i, 0)),
        out_shape=jax.ShapeDtypeStruct((N, D1), jnp.float32),
    )(s0, s1, s2, s3, dii, dio, b0, W1, z1)


def _final_body(p_ref, dii_ref, b1_ref, y_ref):
    p = p_ref[0] + p_ref[1]
    y_ref[...] = p[:, :C] * dii_ref[...] + b1_ref[...] + p[:, C:]


def _final_call(parts, dii, b1):
    return pl.pallas_call(
        _final_body,
        grid=(N // RB,),
        in_specs=[
            pl.BlockSpec((NC, RB, D1), lambda i: (0, i, 0)),
            pl.BlockSpec((RB, 1), lambda i: (i, 0)),
            pl.BlockSpec((1, C), lambda i: (0, 0)),
        ],
        out_specs=pl.BlockSpec((RB, C), lambda i: (i, 0)),
        out_shape=jax.ShapeDtypeStruct((N, C), jnp.float32),
    )(parts, dii, b1)


def kernel(raw_x, edge_index, W0, b0, W1, b1, R0, R1):
    src32 = edge_index[0].astype(jnp.int32).reshape(NC * NS, (E // B) // (NC * NS), B)
    dst32 = edge_index[1].astype(jnp.int32).reshape(NC * NS, (E // B) // (NC * NS), B)

    counts = _deg_call(src32, dst32, jnp.zeros((N, 16), jnp.float32))
    u, f2, f3, z1 = _pre_a_call(raw_x, W0, R0, R1)
    f0, f1, dio, dii = _pre_b_call(counts, u)
    s0, s1, s2, s3 = _spmm0_call(f0, f1, f2, f3, src32, dst32,
                                 jnp.zeros((N, FH), jnp.float32))
    cat1 = _mid_call(s0, s1, s2, s3, dii, dio, b0.reshape(1, F), W1, z1)
    parts = _spmm1_call(cat1, src32, dst32, jnp.zeros((N, D1), jnp.float32))
    y = _final_call(parts, dii, b1.reshape(1, C))
    return y
